# Initial kernel scaffold; baseline (speedup 1.0000x reference)
import functools

import jax
import jax.numpy as jnp
from jax import lax
from jax.experimental import pallas as pl
from jax.experimental.pallas import tpu as pltpu
from jax.experimental.pallas import tpu_sc as plsc

N = 10000
NP = 10240
E = 320000
NSC = 2
NSUB = 16
NTILES = NSC * NSUB
CHUNK = 128
PT = 10112
EP = PT * NTILES
NCHUNK = PT // CHUNK
RPT = NP // NSUB
BLK = 256
EPS = 1e-5
BNSCALE = 1.0 / float(jnp.sqrt(jnp.float32(1.0 + EPS)))
NEG = -3.4e38

_MESH = plsc.VectorSubcoreMesh(core_axis_name="c", subcore_axis_name="s")


def _make_deg():
    @functools.partial(
        pl.kernel,
        out_type=jax.ShapeDtypeStruct((NSC, NP, 16), jnp.float32),
        mesh=_MESH,
        scratch_types=[
            pltpu.VMEM((CHUNK,), jnp.int32),
            pltpu.VMEM((CHUNK, 16), jnp.float32),
            pltpu.VMEM_SHARED((NP, 16), jnp.float32),
        ],
    )
    def deg_k(dst_hbm, zeros_hbm, ones_hbm, out_hbm, didx, ones_v, dacc):
        c = lax.axis_index("c")
        s = lax.axis_index("s")
        r0 = s * RPT
        pltpu.sync_copy(ones_hbm, ones_v)
        pltpu.sync_copy(zeros_hbm.at[pl.ds(r0, RPT)], dacc.at[pl.ds(r0, RPT)])
        plsc.subcore_barrier()
        ebase = (c * NSUB + s) * PT

        def body(i, carry):
            off = ebase + i * CHUNK
            pltpu.sync_copy(dst_hbm.at[pl.ds(off, CHUNK)], didx)
            pltpu.sync_copy(ones_v, dacc.at[didx], add=True)
            return carry

        lax.fori_loop(0, NCHUNK, body, 0)
        plsc.subcore_barrier()
        pltpu.sync_copy(dacc.at[pl.ds(r0, RPT)], out_hbm.at[c, pl.ds(r0, RPT)])

    return deg_k


def _make_agg(F):
    @functools.partial(
        pl.kernel,
        out_type=jax.ShapeDtypeStruct((NSC, NP, F), jnp.float32),
        mesh=_MESH,
        scratch_types=[
            pltpu.VMEM((CHUNK,), jnp.int32),
            pltpu.VMEM((CHUNK,), jnp.int32),
            pltpu.VMEM((CHUNK, F), jnp.float32),
            pltpu.VMEM_SHARED((NP, F), jnp.float32),
            pltpu.SemaphoreType.DMA,
        ],
    )
    def agg_k(src_hbm, dst_hbm, hs_hbm, zeros_hbm, out_hbm,
              sidx, didx, rows, acc, sem):
        c = lax.axis_index("c")
        s = lax.axis_index("s")
        r0 = s * RPT
        pltpu.sync_copy(zeros_hbm.at[pl.ds(r0, RPT)], acc.at[pl.ds(r0, RPT)])
        plsc.subcore_barrier()
        ebase = (c * NSUB + s) * PT

        def body(i, carry):
            off = ebase + i * CHUNK
            pltpu.sync_copy(src_hbm.at[pl.ds(off, CHUNK)], sidx)
            pltpu.sync_copy(dst_hbm.at[pl.ds(off, CHUNK)], didx)
            pltpu.async_copy(hs_hbm.at[sidx], rows, sem).wait()
            pltpu.sync_copy(rows, acc.at[didx], add=True)
            return carry

        lax.fori_loop(0, NCHUNK, body, 0)
        plsc.subcore_barrier()
        pltpu.sync_copy(acc.at[pl.ds(r0, RPT)], out_hbm.at[c, pl.ds(r0, RPT)])

    return agg_k


_DEG = _make_deg()
_AGG128 = _make_agg(128)
_AGG64 = _make_agg(64)


def _rowmask(pid):
    row = pid * BLK + lax.broadcasted_iota(jnp.int32, (BLK, 1), 0)
    return (row < N).astype(jnp.float32)


def _tc1(xp, W1, degp):
    def body(x_ref, w_ref, d_ref, hs_ref, dinv_ref):
        pid = pl.program_id(0)
        dd = d_ref[...]
        dinv = lax.rsqrt(1.0 + dd[0, :, 0:1] + dd[1, :, 0:1])
        h = jnp.dot(x_ref[...], w_ref[...], preferred_element_type=jnp.float32)
        hs_ref[...] = h * dinv * _rowmask(pid)
        dinv_ref[...] = dinv

    return pl.pallas_call(
        body,
        grid=(NP // BLK,),
        in_specs=[
            pl.BlockSpec((BLK, 128), lambda i: (i, 0)),
            pl.BlockSpec((128, 128), lambda i: (0, 0)),
            pl.BlockSpec((NSC, BLK, 16), lambda i: (0, i, 0)),
        ],
        out_specs=[
            pl.BlockSpec((BLK, 128), lambda i: (i, 0)),
            pl.BlockSpec((BLK, 1), lambda i: (i, 0)),
        ],
        out_shape=[
            jax.ShapeDtypeStruct((NP, 128), jnp.float32),
            jax.ShapeDtypeStruct((NP, 1), jnp.float32),
        ],
    )(xp, W1, degp)


def _tc_mid(acc, hs, dinv, b, g, be, W, fout):
    def body(acc_ref, hs_ref, dinv_ref, b_ref, g_ref, be_ref, w_ref, o_ref):
        pid = pl.program_id(0)
        a = acc_ref[...]
        di = dinv_ref[...]
        z = (a[0] + a[1] + hs_ref[...]) * di + b_ref[...][None, :]
        z = z * (g_ref[...] * BNSCALE)[None, :] + be_ref[...][None, :]
        z = jnp.maximum(z, 0.0) * _rowmask(pid)
        h = jnp.dot(z, w_ref[...], preferred_element_type=jnp.float32)
        o_ref[...] = h * di

    return pl.pallas_call(
        body,
        grid=(NP // BLK,),
        in_specs=[
            pl.BlockSpec((NSC, BLK, 128), lambda i: (0, i, 0)),
            pl.BlockSpec((BLK, 128), lambda i: (i, 0)),
            pl.BlockSpec((BLK, 1), lambda i: (i, 0)),
            pl.BlockSpec((128,), lambda i: (0,)),
            pl.BlockSpec((128,), lambda i: (0,)),
            pl.BlockSpec((128,), lambda i: (0,)),
            pl.BlockSpec((128, fout), lambda i: (0, 0)),
        ],
        out_specs=pl.BlockSpec((BLK, fout), lambda i: (i, 0)),
        out_shape=jax.ShapeDtypeStruct((NP, fout), jnp.float32),
    )(acc, hs, dinv, b, g, be, W)


def _tc_out(acc, hs, dinv, b):
    def body(acc_ref, hs_ref, dinv_ref, b_ref, o_ref):
        a = acc_ref[...]
        z = (a[0] + a[1] + hs_ref[...]) * dinv_ref[...] + b_ref[...][None, :]
        col = lax.broadcasted_iota(jnp.int32, (BLK, 64), 1)
        valid = col < 40
        zm = jnp.where(valid, z, NEG)
        m = jnp.max(zm, axis=1, keepdims=True)
        e = jnp.where(valid, jnp.exp(z - m), 0.0)
        ssum = jnp.sum(e, axis=1, keepdims=True)
        o_ref[...] = z - m - jnp.log(ssum)

    return pl.pallas_call(
        body,
        grid=(NP // BLK,),
        in_specs=[
            pl.BlockSpec((NSC, BLK, 64), lambda i: (0, i, 0)),
            pl.BlockSpec((BLK, 64), lambda i: (i, 0)),
            pl.BlockSpec((BLK, 1), lambda i: (i, 0)),
            pl.BlockSpec((64,), lambda i: (0,)),
        ],
        out_specs=pl.BlockSpec((BLK, 64), lambda i: (i, 0)),
        out_shape=jax.ShapeDtypeStruct((NP, 64), jnp.float32),
    )(acc, hs, dinv, b)


def kernel(x, edge_index, W1, b1, g1, be1, W2, b2, g2, be2, W3, b3):
    ei = edge_index.astype(jnp.int32)
    pad_e = jnp.full((EP - E,), N, jnp.int32)
    src = jnp.concatenate([ei[0], pad_e])
    dst = jnp.concatenate([ei[1], pad_e])
    xp = jnp.pad(x, ((0, NP - N), (0, 0)))
    W3p = jnp.pad(W3, ((0, 0), (0, 64 - 40)))
    b3p = jnp.pad(b3, (0, 64 - 40))
    zeros16 = jnp.zeros((NP, 16), jnp.float32)
    ones16 = jnp.ones((CHUNK, 16), jnp.float32)
    zeros128 = jnp.zeros((NP, 128), jnp.float32)
    zeros64 = jnp.zeros((NP, 64), jnp.float32)

    degp = _DEG(dst, zeros16, ones16)
    hs1, dinv = _tc1(xp, W1, degp)
    acc1 = _AGG128(src, dst, hs1, zeros128)
    hs2 = _tc_mid(acc1, hs1, dinv, b1, g1, be1, W2, 128)
    acc2 = _AGG128(src, dst, hs2, zeros128)
    hs3 = _tc_mid(acc2, hs2, dinv, b2, g2, be2, W3p, 64)
    acc3 = _AGG64(src, dst, hs3, zeros64)
    outp = _tc_out(acc3, hs3, dinv, b3p)
    return outp[:N, :40]


# speedup vs baseline: 7.9106x; 7.9106x over previous
import functools
import math

import jax
import jax.numpy as jnp
from jax import lax
from jax.experimental import pallas as pl
from jax.experimental.pallas import tpu as pltpu
from jax.experimental.pallas import tpu_sc as plsc

N = 10000
NP = 10240
E = 320000
NSC = 2
NSUB = 16
NTILES = NSC * NSUB
CHUNK = 128
PT = 10112
EP = PT * NTILES
NCHUNK = PT // CHUNK
RPT = NP // NSUB
BLK = 256
EPS = 1e-5
BNSCALE = 1.0 / math.sqrt(1.0 + EPS)
NEG = -3.4e38

_MESH = plsc.VectorSubcoreMesh(core_axis_name="c", subcore_axis_name="s")


def _make_deg():
    @functools.partial(
        pl.kernel,
        out_type=jax.ShapeDtypeStruct((NSC, NP, 128), jnp.float32),
        mesh=_MESH,
        scratch_types=[
            pltpu.VMEM((CHUNK,), jnp.int32),
            pltpu.VMEM((CHUNK, 128), jnp.float32),
            pltpu.VMEM_SHARED((NP, 128), jnp.float32),
        ],
    )
    def deg_k(dst_hbm, zeros_hbm, ones_hbm, out_hbm, didx, ones_v, dacc):
        c = lax.axis_index("c")
        s = lax.axis_index("s")
        r0 = s * RPT
        pltpu.sync_copy(ones_hbm, ones_v)
        pltpu.sync_copy(zeros_hbm.at[pl.ds(r0, RPT)], dacc.at[pl.ds(r0, RPT)])
        plsc.subcore_barrier()
        ebase = (c * NSUB + s) * PT

        def body(i, carry):
            off = ebase + i * CHUNK
            pltpu.sync_copy(dst_hbm.at[pl.ds(off, CHUNK)], didx)
            pltpu.sync_copy(ones_v, dacc.at[didx], add=True)
            return carry

        lax.fori_loop(0, NCHUNK, body, 0)
        plsc.subcore_barrier()
        pltpu.sync_copy(dacc.at[pl.ds(r0, RPT)], out_hbm.at[c, pl.ds(r0, RPT)])

    return deg_k


def _make_agg(F):
    @functools.partial(
        pl.kernel,
        out_type=jax.ShapeDtypeStruct((NSC, NP, F), jnp.float32),
        mesh=_MESH,
        scratch_types=[
            pltpu.VMEM((CHUNK,), jnp.int32),
            pltpu.VMEM((CHUNK,), jnp.int32),
            pltpu.VMEM((CHUNK, F), jnp.float32),
            pltpu.VMEM_SHARED((NP, F), jnp.float32),
            pltpu.SemaphoreType.DMA,
        ],
    )
    def agg_k(src_hbm, dst_hbm, hs_hbm, zeros_hbm, out_hbm,
              sidx, didx, rows, acc, sem):
        c = lax.axis_index("c")
        s = lax.axis_index("s")
        r0 = s * RPT
        pltpu.sync_copy(zeros_hbm.at[pl.ds(r0, RPT)], acc.at[pl.ds(r0, RPT)])
        plsc.subcore_barrier()
        ebase = (c * NSUB + s) * PT

        def body(i, carry):
            off = ebase + i * CHUNK
            pltpu.sync_copy(src_hbm.at[pl.ds(off, CHUNK)], sidx)
            pltpu.sync_copy(dst_hbm.at[pl.ds(off, CHUNK)], didx)
            pltpu.async_copy(hs_hbm.at[sidx], rows, sem).wait()
            pltpu.sync_copy(rows, acc.at[didx], add=True)
            return carry

        lax.fori_loop(0, NCHUNK, body, 0)
        plsc.subcore_barrier()
        pltpu.sync_copy(acc.at[pl.ds(r0, RPT)], out_hbm.at[c, pl.ds(r0, RPT)])

    return agg_k


_DEG = _make_deg()
_AGG128 = _make_agg(128)


def _rowmask(pid):
    row = pid * BLK + lax.broadcasted_iota(jnp.int32, (BLK, 1), 0)
    return (row < N).astype(jnp.float32)


def _tc1(xp, W1, degp):
    def body(x_ref, w_ref, d_ref, hs_ref, dinv_ref):
        pid = pl.program_id(0)
        dd = d_ref[...]
        dinv = lax.rsqrt(1.0 + dd[0, :, 0:1] + dd[1, :, 0:1])
        h = jnp.dot(x_ref[...], w_ref[...], preferred_element_type=jnp.float32)
        hs_ref[...] = h * dinv * _rowmask(pid)
        dinv_ref[...] = dinv

    return pl.pallas_call(
        body,
        grid=(NP // BLK,),
        in_specs=[
            pl.BlockSpec((BLK, 128), lambda i: (i, 0)),
            pl.BlockSpec((128, 128), lambda i: (0, 0)),
            pl.BlockSpec((NSC, BLK, 128), lambda i: (0, i, 0)),
        ],
        out_specs=[
            pl.BlockSpec((BLK, 128), lambda i: (i, 0)),
            pl.BlockSpec((BLK, 1), lambda i: (i, 0)),
        ],
        out_shape=[
            jax.ShapeDtypeStruct((NP, 128), jnp.float32),
            jax.ShapeDtypeStruct((NP, 1), jnp.float32),
        ],
    )(xp, W1, degp)


def _tc_mid(acc, hs, dinv, b, g, be, W, fout):
    def body(acc_ref, hs_ref, dinv_ref, b_ref, g_ref, be_ref, w_ref, o_ref):
        pid = pl.program_id(0)
        a = acc_ref[...]
        di = dinv_ref[...]
        z = (a[0] + a[1] + hs_ref[...]) * di + b_ref[...][None, :]
        z = z * (g_ref[...] * BNSCALE)[None, :] + be_ref[...][None, :]
        z = jnp.maximum(z, 0.0) * _rowmask(pid)
        h = jnp.dot(z, w_ref[...], preferred_element_type=jnp.float32)
        o_ref[...] = h * di

    return pl.pallas_call(
        body,
        grid=(NP // BLK,),
        in_specs=[
            pl.BlockSpec((NSC, BLK, 128), lambda i: (0, i, 0)),
            pl.BlockSpec((BLK, 128), lambda i: (i, 0)),
            pl.BlockSpec((BLK, 1), lambda i: (i, 0)),
            pl.BlockSpec((128,), lambda i: (0,)),
            pl.BlockSpec((128,), lambda i: (0,)),
            pl.BlockSpec((128,), lambda i: (0,)),
            pl.BlockSpec((128, fout), lambda i: (0, 0)),
        ],
        out_specs=pl.BlockSpec((BLK, fout), lambda i: (i, 0)),
        out_shape=jax.ShapeDtypeStruct((NP, fout), jnp.float32),
    )(acc, hs, dinv, b, g, be, W)


def _tc_out(acc, hs, dinv, b):
    def body(acc_ref, hs_ref, dinv_ref, b_ref, o_ref):
        a = acc_ref[...]
        z = (a[0] + a[1] + hs_ref[...]) * dinv_ref[...] + b_ref[...][None, :]
        col = lax.broadcasted_iota(jnp.int32, (BLK, 128), 1)
        valid = col < 40
        zm = jnp.where(valid, z, NEG)
        m = jnp.max(zm, axis=1, keepdims=True)
        e = jnp.where(valid, jnp.exp(z - m), 0.0)
        ssum = jnp.sum(e, axis=1, keepdims=True)
        o_ref[...] = z - m - jnp.log(ssum)

    return pl.pallas_call(
        body,
        grid=(NP // BLK,),
        in_specs=[
            pl.BlockSpec((NSC, BLK, 128), lambda i: (0, i, 0)),
            pl.BlockSpec((BLK, 128), lambda i: (i, 0)),
            pl.BlockSpec((BLK, 1), lambda i: (i, 0)),
            pl.BlockSpec((128,), lambda i: (0,)),
        ],
        out_specs=pl.BlockSpec((BLK, 128), lambda i: (i, 0)),
        out_shape=jax.ShapeDtypeStruct((NP, 128), jnp.float32),
    )(acc, hs, dinv, b)


def kernel(x, edge_index, W1, b1, g1, be1, W2, b2, g2, be2, W3, b3):
    ei = edge_index.astype(jnp.int32)
    pad_e = jnp.full((EP - E,), N, jnp.int32)
    src = jnp.concatenate([ei[0], pad_e])
    dst = jnp.concatenate([ei[1], pad_e])
    xp = jnp.pad(x, ((0, NP - N), (0, 0)))
    W3p = jnp.pad(W3, ((0, 0), (0, 128 - 40)))
    b3p = jnp.pad(b3, (0, 128 - 40))
    zeros128 = jnp.zeros((NP, 128), jnp.float32)
    ones128 = jnp.ones((CHUNK, 128), jnp.float32)

    degp = _DEG(dst, zeros128, ones128)
    hs1, dinv = _tc1(xp, W1, degp)
    acc1 = _AGG128(src, dst, hs1, zeros128)
    hs2 = _tc_mid(acc1, hs1, dinv, b1, g1, be1, W2, 128)
    acc2 = _AGG128(src, dst, hs2, zeros128)
    hs3 = _tc_mid(acc2, hs2, dinv, b2, g2, be2, W3p, 128)
    acc3 = _AGG128(src, dst, hs3, zeros128)
    outp = _tc_out(acc3, hs3, dinv, b3p)
    return outp[:N, :40]


# speedup vs baseline: 8.4389x; 1.0668x over previous
import functools
import math

import jax
import jax.numpy as jnp
from jax import lax
from jax.experimental import pallas as pl
from jax.experimental.pallas import tpu as pltpu
from jax.experimental.pallas import tpu_sc as plsc

N = 10000
NP = 10240
E = 320000
NSC = 2
NSUB = 16
NTILES = NSC * NSUB
CHUNK = 128
NBUF = 2
PT = 10240
EP = PT * NTILES
NCHUNK = PT // CHUNK
NPH = 2
CPH = NCHUNK // NPH
GPH = CPH // NBUF
RPT = NP // NSUB
BLK = 256
EPS = 1e-5
BNSCALE = 1.0 / math.sqrt(1.0 + EPS)
NEG = -3.4e38

_MESH = plsc.VectorSubcoreMesh(core_axis_name="c", subcore_axis_name="s")


def _make_deg():
    @functools.partial(
        pl.kernel,
        out_type=jax.ShapeDtypeStruct((NSC, NP, 128), jnp.float32),
        mesh=_MESH,
        scratch_types=[
            pltpu.VMEM((NCHUNK, CHUNK), jnp.int32),
            pltpu.VMEM((CHUNK, 128), jnp.float32),
            pltpu.VMEM_SHARED((NP, 128), jnp.float32),
            pltpu.SemaphoreType.DMA,
        ],
    )
    def deg_k(dst_hbm, zeros_hbm, ones_hbm, out_hbm, didx_all, ones_v, dacc,
              ssem):
        c = lax.axis_index("c")
        s = lax.axis_index("s")
        wid = c * NSUB + s
        r0 = s * RPT
        pltpu.sync_copy(ones_hbm, ones_v)
        pltpu.sync_copy(dst_hbm.at[wid], didx_all)
        pltpu.sync_copy(zeros_hbm.at[pl.ds(r0, RPT)], dacc.at[pl.ds(r0, RPT)])
        plsc.subcore_barrier()

        def body(j, carry):
            pltpu.async_copy(ones_v, dacc.at[didx_all.at[j]], ssem, add=True)
            return carry

        lax.fori_loop(0, NCHUNK, body, 0)

        def drain(j, carry):
            pltpu.make_async_copy(ones_hbm, ones_v, ssem).wait()
            return carry

        lax.fori_loop(0, NCHUNK, drain, 0)
        plsc.subcore_barrier()
        pltpu.sync_copy(dacc.at[pl.ds(r0, RPT)], out_hbm.at[c, pl.ds(r0, RPT)])

    return deg_k


def _make_agg(F):
    @functools.partial(
        pl.kernel,
        out_type=jax.ShapeDtypeStruct((NSC, NP, F), jnp.float32),
        mesh=_MESH,
        scratch_types=[
            pltpu.VMEM((CPH, CHUNK), jnp.int32),
            pltpu.VMEM((CPH, CHUNK), jnp.int32),
            pltpu.VMEM((NBUF, CHUNK, F), jnp.float32),
            pltpu.VMEM_SHARED((NP, F), jnp.float32),
        ] + [pltpu.SemaphoreType.DMA] * (2 * NBUF),
    )
    def agg_k(src_hbm, dst_hbm, hs_hbm, zeros_hbm, out_hbm,
              sidx_h, didx_h, rows, acc, *sems):
        gsem = sems[:NBUF]
        ssem = sems[NBUF:]
        c = lax.axis_index("c")
        s = lax.axis_index("s")
        wid = c * NSUB + s
        r0 = s * RPT
        pltpu.sync_copy(zeros_hbm.at[pl.ds(r0, RPT)], acc.at[pl.ds(r0, RPT)])
        plsc.subcore_barrier()

        for p in range(NPH):
            pltpu.sync_copy(src_hbm.at[wid, pl.ds(p * CPH, CPH)], sidx_h)
            pltpu.sync_copy(dst_hbm.at[wid, pl.ds(p * CPH, CPH)], didx_h)
            for b in range(NBUF):
                pltpu.async_copy(hs_hbm.at[sidx_h.at[b]], rows.at[b], gsem[b])

            def body(g, carry):
                scat = []
                for b in range(NBUF):
                    j = g * NBUF + b
                    pltpu.make_async_copy(
                        hs_hbm.at[pl.ds(0, CHUNK)], rows.at[b], gsem[b]).wait()
                    scat.append(pltpu.async_copy(
                        rows.at[b], acc.at[didx_h.at[j]], ssem[b], add=True))
                for b in range(NBUF):
                    scat[b].wait()
                    j2 = (g + 1) * NBUF + b

                    @pl.when(j2 < CPH)
                    def _():
                        pltpu.async_copy(
                            hs_hbm.at[sidx_h.at[j2]], rows.at[b], gsem[b])
                return carry

            lax.fori_loop(0, GPH, body, 0)

        plsc.subcore_barrier()
        pltpu.sync_copy(acc.at[pl.ds(r0, RPT)], out_hbm.at[c, pl.ds(r0, RPT)])

    return agg_k


_DEG = _make_deg()
_AGG128 = _make_agg(128)


def _rowmask(pid):
    row = pid * BLK + lax.broadcasted_iota(jnp.int32, (BLK, 1), 0)
    return (row < N).astype(jnp.float32)


def _tc1(xp, W1, degp):
    def body(x_ref, w_ref, d_ref, hs_ref, dinv_ref):
        pid = pl.program_id(0)
        dd = d_ref[...]
        dinv = lax.rsqrt(1.0 + dd[0, :, 0:1] + dd[1, :, 0:1])
        h = jnp.dot(x_ref[...], w_ref[...], preferred_element_type=jnp.float32)
        hs_ref[...] = h * dinv * _rowmask(pid)
        dinv_ref[...] = dinv

    return pl.pallas_call(
        body,
        grid=(NP // BLK,),
        in_specs=[
            pl.BlockSpec((BLK, 128), lambda i: (i, 0)),
            pl.BlockSpec((128, 128), lambda i: (0, 0)),
            pl.BlockSpec((NSC, BLK, 128), lambda i: (0, i, 0)),
        ],
        out_specs=[
            pl.BlockSpec((BLK, 128), lambda i: (i, 0)),
            pl.BlockSpec((BLK, 1), lambda i: (i, 0)),
        ],
        out_shape=[
            jax.ShapeDtypeStruct((NP, 128), jnp.float32),
            jax.ShapeDtypeStruct((NP, 1), jnp.float32),
        ],
    )(xp, W1, degp)


def _tc_mid(acc, hs, dinv, b, g, be, W, fout):
    def body(acc_ref, hs_ref, dinv_ref, b_ref, g_ref, be_ref, w_ref, o_ref):
        pid = pl.program_id(0)
        a = acc_ref[...]
        di = dinv_ref[...]
        z = (a[0] + a[1] + hs_ref[...]) * di + b_ref[...][None, :]
        z = z * (g_ref[...] * BNSCALE)[None, :] + be_ref[...][None, :]
        z = jnp.maximum(z, 0.0) * _rowmask(pid)
        h = jnp.dot(z, w_ref[...], preferred_element_type=jnp.float32)
        o_ref[...] = h * di

    return pl.pallas_call(
        body,
        grid=(NP // BLK,),
        in_specs=[
            pl.BlockSpec((NSC, BLK, 128), lambda i: (0, i, 0)),
            pl.BlockSpec((BLK, 128), lambda i: (i, 0)),
            pl.BlockSpec((BLK, 1), lambda i: (i, 0)),
            pl.BlockSpec((128,), lambda i: (0,)),
            pl.BlockSpec((128,), lambda i: (0,)),
            pl.BlockSpec((128,), lambda i: (0,)),
            pl.BlockSpec((128, fout), lambda i: (0, 0)),
        ],
        out_specs=pl.BlockSpec((BLK, fout), lambda i: (i, 0)),
        out_shape=jax.ShapeDtypeStruct((NP, fout), jnp.float32),
    )(acc, hs, dinv, b, g, be, W)


def _tc_out(acc, hs, dinv, b):
    def body(acc_ref, hs_ref, dinv_ref, b_ref, o_ref):
        a = acc_ref[...]
        z = (a[0] + a[1] + hs_ref[...]) * dinv_ref[...] + b_ref[...][None, :]
        col = lax.broadcasted_iota(jnp.int32, (BLK, 128), 1)
        valid = col < 40
        zm = jnp.where(valid, z, NEG)
        m = jnp.max(zm, axis=1, keepdims=True)
        e = jnp.where(valid, jnp.exp(z - m), 0.0)
        ssum = jnp.sum(e, axis=1, keepdims=True)
        o_ref[...] = z - m - jnp.log(ssum)

    return pl.pallas_call(
        body,
        grid=(NP // BLK,),
        in_specs=[
            pl.BlockSpec((NSC, BLK, 128), lambda i: (0, i, 0)),
            pl.BlockSpec((BLK, 128), lambda i: (i, 0)),
            pl.BlockSpec((BLK, 1), lambda i: (i, 0)),
            pl.BlockSpec((128,), lambda i: (0,)),
        ],
        out_specs=pl.BlockSpec((BLK, 128), lambda i: (i, 0)),
        out_shape=jax.ShapeDtypeStruct((NP, 128), jnp.float32),
    )(acc, hs, dinv, b)


def kernel(x, edge_index, W1, b1, g1, be1, W2, b2, g2, be2, W3, b3):
    ei = edge_index.astype(jnp.int32)
    pad_e = jnp.full((EP - E,), N, jnp.int32)
    src = jnp.concatenate([ei[0], pad_e]).reshape(NTILES, NCHUNK, CHUNK)
    dst = jnp.concatenate([ei[1], pad_e]).reshape(NTILES, NCHUNK, CHUNK)
    xp = jnp.pad(x, ((0, NP - N), (0, 0)))
    W3p = jnp.pad(W3, ((0, 0), (0, 128 - 40)))
    b3p = jnp.pad(b3, (0, 128 - 40))
    zeros128 = jnp.zeros((NP, 128), jnp.float32)
    ones128 = jnp.ones((CHUNK, 128), jnp.float32)

    degp = _DEG(dst, zeros128, ones128)
    hs1, dinv = _tc1(xp, W1, degp)
    acc1 = _AGG128(src, dst, hs1, zeros128)
    hs2 = _tc_mid(acc1, hs1, dinv, b1, g1, be1, W2, 128)
    acc2 = _AGG128(src, dst, hs2, zeros128)
    hs3 = _tc_mid(acc2, hs2, dinv, b2, g2, be2, W3p, 128)
    acc3 = _AGG128(src, dst, hs3, zeros128)
    outp = _tc_out(acc3, hs3, dinv, b3p)
    return outp[:N, :40]
